# baseline (speedup 1.0000x reference)
import functools

import jax
import jax.numpy as jnp
from jax import lax
from jax.experimental import pallas as pl
from jax.experimental.pallas import tpu as pltpu
from jax.experimental.pallas import tpu_sc as plsc

L = 16
TOK = 128
NG = TOK // L


def _make_lookup(J, T, D):
    info = plsc.get_sparse_core_info()
    NC, NS = info.num_cores, info.num_subcores
    NW = NC * NS
    assert T % (NW * TOK) == 0 or T == NW * TOK
    mesh = plsc.VectorSubcoreMesh(core_axis_name="c", subcore_axis_name="s")

    @functools.partial(
        pl.kernel,
        mesh=mesh,
        out_type=jax.ShapeDtypeStruct((J, D, T), jnp.float32),
        scratch_types=[
            pltpu.VMEM((J, TOK), jnp.int32),
            pltpu.VMEM((TOK,), jnp.int32),
            pltpu.VMEM((TOK,), jnp.int32),
            pltpu.VMEM((TOK, 2 * D), jnp.float32),
            pltpu.VMEM((TOK, 2 * D), jnp.float32),
            pltpu.VMEM((D, TOK), jnp.float32),
            pltpu.VMEM((D, TOK), jnp.float32),
            pltpu.SemaphoreType.DMA,
            pltpu.SemaphoreType.DMA,
            pltpu.SemaphoreType.DMA,
            pltpu.SemaphoreType.DMA,
        ],
        compiler_params=pltpu.CompilerParams(needs_layout_passes=False),
    )
    def lookup(xt_hbm, tab_hbm, out_hbm, idx_all, i20, i21, buf0, buf1,
               st0, st1, sg0, sg1, sw0, sw1):
        wid = lax.axis_index("s") * NC + lax.axis_index("c")
        col0 = pl.multiple_of(wid * TOK, TOK)
        pltpu.sync_copy(xt_hbm.at[:, pl.ds(col0, TOK)], idx_all)

        idx2 = (i20, i21)
        bufs = (buf0, buf1)
        stag = (st0, st1)
        sg = (sg0, sg1)
        sw = (sw0, sw1)

        def compute_idx2(j, d):
            for g in range(NG):
                v = idx_all[j, pl.ds(L * g, L)]
                idx2[d][pl.ds(L * g, L)] = lax.shift_right_logical(v, 1)

        def fire_gather(d):
            pltpu.async_copy(tab_hbm.at[idx2[d]], bufs[d], sg[d])

        def wait_gather(d):
            pltpu.make_async_copy(tab_hbm.at[idx2[d]], bufs[d], sg[d]).wait()

        def select(j, d):
            for g in range(NG):
                v = idx_all[j, pl.ds(L * g, L)]
                sel = (v & 1) * D
                trow = jnp.arange(L, dtype=jnp.int32) + (L * g)
                for f in range(D):
                    val = plsc.load_gather(bufs[d], [trow, sel + f])
                    stag[d][f, pl.ds(L * g, L)] = val

        def fire_write(j, d):
            pltpu.async_copy(stag[d], out_hbm.at[j, :, pl.ds(col0, TOK)], sw[d])

        def wait_write(j, d):
            pltpu.make_async_copy(
                stag[d], out_hbm.at[j, :, pl.ds(col0, TOK)], sw[d]
            ).wait()

        compute_idx2(0, 0)
        fire_gather(0)

        def pair(p, carry):
            for b in range(2):
                j = 2 * p + b

                @pl.when(j < J - 1)
                def _():
                    compute_idx2(j + 1, 1 - b)
                    fire_gather(1 - b)

                wait_gather(b)

                @pl.when(j >= 2)
                def _():
                    wait_write(j - 2, b)

                select(j, b)
                fire_write(j, b)
            return carry

        lax.fori_loop(0, J // 2, pair, 0)
        wait_write(J - 2, 0)
        wait_write(J - 1, 1)

    return lookup


def kernel(x, table):
    T, J = x.shape
    V, D = table.shape
    xt = x.T.astype(jnp.int32)
    tab2 = table.reshape(V // 2, 2 * D)
    out3 = _make_lookup(J, T, D)(xt, tab2)
    return jnp.transpose(out3, (2, 0, 1))


# speedup vs baseline: 1.8595x; 1.8595x over previous
import functools

import jax
import jax.numpy as jnp
from jax import lax
from jax.experimental import pallas as pl
from jax.experimental.pallas import tpu as pltpu
from jax.experimental.pallas import tpu_sc as plsc

L = 16
TOK = 128
NG = TOK // L


def _make_lookup(J, T, D):
    info = plsc.get_sparse_core_info()
    NC, NS = info.num_cores, info.num_subcores
    NW = NC * NS
    assert T % (NW * TOK) == 0 or T == NW * TOK
    mesh = plsc.VectorSubcoreMesh(core_axis_name="c", subcore_axis_name="s")

    @functools.partial(
        pl.kernel,
        mesh=mesh,
        out_type=jax.ShapeDtypeStruct((J, D, T), jnp.float32),
        scratch_types=[
            pltpu.VMEM((J, TOK), jnp.int32),
            pltpu.VMEM((TOK,), jnp.int32),
            pltpu.VMEM((TOK,), jnp.int32),
            pltpu.VMEM((TOK, 2 * D), jnp.float32),
            pltpu.VMEM((TOK, 2 * D), jnp.float32),
            pltpu.VMEM((D, TOK), jnp.float32),
            pltpu.VMEM((D, TOK), jnp.float32),
            pltpu.SemaphoreType.DMA,
            pltpu.SemaphoreType.DMA,
            pltpu.SemaphoreType.DMA,
            pltpu.SemaphoreType.DMA,
        ],
        compiler_params=pltpu.CompilerParams(needs_layout_passes=False),
    )
    def lookup(xt_hbm, tab_hbm, out_hbm, idx_all, i20, i21, buf0, buf1,
               st0, st1, sg0, sg1, sw0, sw1):
        wid = lax.axis_index("s") * NC + lax.axis_index("c")
        col0 = pl.multiple_of(wid * TOK, TOK)
        pltpu.sync_copy(xt_hbm.at[:, pl.ds(col0, TOK)], idx_all)

        idx2 = (i20, i21)
        bufs = (buf0, buf1)
        stag = (st0, st1)
        sg = (sg0, sg1)
        sw = (sw0, sw1)

        def compute_idx2(j, d):
            for g in range(NG):
                v = idx_all[j, pl.ds(L * g, L)]
                idx2[d][pl.ds(L * g, L)] = lax.shift_right_logical(v, 1)

        def fire_gather(d):
            pltpu.async_copy(tab_hbm.at[idx2[d]], bufs[d], sg[d])

        def wait_gather(d):
            pltpu.make_async_copy(tab_hbm.at[idx2[d]], bufs[d], sg[d]).wait()

        def select(j, d):
            iota = jnp.arange(L, dtype=jnp.int32)
            for g in range(NG):
                v = idx_all[j, pl.ds(L * g, L)]
                sel = (v & 1) * D
                trow = iota + (L * g)
                def diag(k, c):
                    perm = (iota + k) & (L - 1)
                    for fb in range(D // L):
                        frow = perm + (L * fb)
                        val = plsc.load_gather(bufs[d], [trow, sel + frow])
                        plsc.store_scatter(stag[d], [frow, trow], val)
                    return c

                lax.fori_loop(0, L, diag, 0)

        def fire_write(j, d):
            pltpu.async_copy(stag[d], out_hbm.at[j, :, pl.ds(col0, TOK)], sw[d])

        def wait_write(j, d):
            pltpu.make_async_copy(
                stag[d], out_hbm.at[j, :, pl.ds(col0, TOK)], sw[d]
            ).wait()

        compute_idx2(0, 0)
        fire_gather(0)

        def pair(p, carry):
            for b in range(2):
                j = 2 * p + b

                @pl.when(j < J - 1)
                def _():
                    compute_idx2(j + 1, 1 - b)
                    fire_gather(1 - b)

                wait_gather(b)

                @pl.when(j >= 2)
                def _():
                    wait_write(j - 2, b)

                select(j, b)
                fire_write(j, b)
            return carry

        lax.fori_loop(0, J // 2, pair, 0)
        wait_write(J - 2, 0)
        wait_write(J - 1, 1)

    return lookup


def kernel(x, table):
    T, J = x.shape
    V, D = table.shape
    xt = x.T.astype(jnp.int32)
    tab2 = table.reshape(V // 2, 2 * D)
    out3 = _make_lookup(J, T, D)(xt, tab2)
    return jnp.transpose(out3, (2, 0, 1))
